# baseline (speedup 1.0000x reference)
import jax
import jax.numpy as jnp
from jax.experimental import pallas as pl
from jax.experimental.pallas import tpu as pltpu

NCLS = 20
KK = 3
OC = (NCLS + 1) * KK * KK
OR = 4 * KK * KK
OSUM = 32
H = 64
W = 64
HW = H * W
STRIDE_LOG2 = 5
CBLK = 512


def _sel_matrices():
    i_c = jax.lax.broadcasted_iota(jnp.int32, (OSUM, OC), 0)
    o_c = jax.lax.broadcasted_iota(jnp.int32, (OSUM, OC), 1)
    s_cls = ((i_c < 21) & (o_c // (KK * KK) == i_c)).astype(jnp.float32)
    i_r = jax.lax.broadcasted_iota(jnp.int32, (OSUM, OR), 0)
    o_r = jax.lax.broadcasted_iota(jnp.int32, (OSUM, OR), 1)
    s_reg = ((i_r >= 21) & (i_r < 25)
             & (o_r // (KK * KK) == i_r - 21)).astype(jnp.float32)
    return s_cls, s_reg


def _proj_kernel(f_ref, wc_ref, wr_ref, bc_ref, br_ref, o_ref):
    cb = pl.program_id(1)
    s_cls, s_reg = _sel_matrices()
    wsum = (jax.lax.dot(s_cls, wc_ref[...], preferred_element_type=jnp.float32)
            + jax.lax.dot(s_reg, wr_ref[...], preferred_element_type=jnp.float32))
    part = jax.lax.dot(wsum.astype(jnp.bfloat16), f_ref[0].astype(jnp.bfloat16),
                       preferred_element_type=jnp.float32)

    @pl.when(cb == 0)
    def _():
        bsum = (jax.lax.dot_general(s_cls, bc_ref[...], (((1,), (1,)), ((), ())),
                                    preferred_element_type=jnp.float32)
                + jax.lax.dot_general(s_reg, br_ref[...], (((1,), (1,)), ((), ())),
                                      preferred_element_type=jnp.float32))
        o_ref[0] = part + bsum

    @pl.when(cb != 0)
    def _():
        o_ref[0] += part


def _pool_kernel(s_ref, p_ref, cls_ref, reg_ref):
    n = p_ref.shape[1]
    pt = jnp.transpose(p_ref[0], (1, 0))
    x1 = pt[0:1, :] >> STRIDE_LOG2
    y1 = pt[1:2, :] >> STRIDE_LOG2
    x2 = (pt[2:3, :] + 31) >> STRIDE_LOG2
    y2 = (pt[3:4, :] + 31) >> STRIDE_LOG2
    third = jnp.float32(1.0 / 3.0)
    hb = jnp.floor((y2 - y1 + 2).astype(jnp.float32) * third).astype(jnp.int32)
    wb = jnp.floor((x2 - x1 + 2).astype(jnp.float32) * third).astype(jnp.int32)
    r = jax.lax.broadcasted_iota(jnp.int32, (H, n), 0)
    rmask = (r >= y1) & (r < y1 + hb)
    cmask = (r >= x1) & (r < x1 + wb)
    rm = rmask.astype(jnp.bfloat16)
    cm = cmask.astype(jnp.bfloat16)
    mask = (rm[:, None, :] * cm[None, :, :]).reshape(HW, n)
    pooled = jax.lax.dot(s_ref[0].astype(jnp.bfloat16), mask,
                         preferred_element_type=jnp.float32)
    denom = (hb * wb).astype(jnp.float32)
    pooled = pooled * (1.0 / denom)
    pot = jnp.transpose(pooled, (1, 0))
    cls_ref[0] = pot[:, 0:21]
    reg_ref[0] = pot[:, 21:25]


@jax.jit
def kernel(features, w_cls, b_cls, w_reg, b_reg, proposals):
    B, Cin, _, _ = features.shape
    N = proposals.shape[1]
    f = features.reshape(B, Cin, HW)

    scores = pl.pallas_call(
        _proj_kernel,
        out_shape=jax.ShapeDtypeStruct((B, OSUM, HW), jnp.float32),
        grid=(B, Cin // CBLK),
        in_specs=[
            pl.BlockSpec((1, CBLK, HW), lambda b, cb: (b, cb, 0)),
            pl.BlockSpec((OC, CBLK), lambda b, cb: (0, cb)),
            pl.BlockSpec((OR, CBLK), lambda b, cb: (0, cb)),
            pl.BlockSpec((1, OC), lambda b, cb: (0, 0)),
            pl.BlockSpec((1, OR), lambda b, cb: (0, 0)),
        ],
        out_specs=pl.BlockSpec((1, OSUM, HW), lambda b, cb: (b, 0, 0)),
        compiler_params=pltpu.CompilerParams(
            dimension_semantics=("parallel", "arbitrary")),
        name="rfcn_proj",
    )(f, w_cls, w_reg, b_cls.reshape(1, OC), b_reg.reshape(1, OR))

    cls_out, reg_out = pl.pallas_call(
        _pool_kernel,
        out_shape=(jax.ShapeDtypeStruct((B, N, 21), jnp.float32),
                   jax.ShapeDtypeStruct((B, N, 4), jnp.float32)),
        grid=(B,),
        in_specs=[
            pl.BlockSpec((1, OSUM, HW), lambda b: (b, 0, 0)),
            pl.BlockSpec((1, N, 4), lambda b: (b, 0, 0)),
        ],
        out_specs=(pl.BlockSpec((1, N, 21), lambda b: (b, 0, 0)),
                   pl.BlockSpec((1, N, 4), lambda b: (b, 0, 0))),
        compiler_params=pltpu.CompilerParams(
            dimension_semantics=("parallel",)),
        name="rfcn_pool",
    )(scores, proposals)

    return cls_out, reg_out
